# baseline (speedup 1.0000x reference)
import jax
import jax.numpy as jnp
from jax.experimental import pallas as pl
from jax.experimental.pallas import tpu as pltpu

T, B, N = 4, 8, 512
BN = B * N
IN_DIM, HID, OUT = 128, 128, 64

BI = 512
BJ = 512
NI = BN // BI
NJ = BN // BJ


def _deg_kernel(m_ref, a_ref, deg_ref):
    i = pl.program_id(1)
    mi = m_ref[0, 0, pl.ds(i * BI, BI)]
    part = jnp.dot(mi[None, :], a_ref[0], preferred_element_type=jnp.float32)

    @pl.when(i == 0)
    def _():
        deg_ref[0] = part

    @pl.when(i > 0)
    def _():
        deg_ref[0] += part


def _g1_kernel(deg_ref, m_ref, x_ref, w1_ref, dis_ref, g_ref):
    m = m_ref[0, 0]
    deg = deg_ref[0, 0] * m + m
    dis = jnp.where(deg > 0, jax.lax.rsqrt(deg), 0.0)
    dis_ref[0, 0] = dis
    h = jax.lax.dot_general(x_ref[0], w1_ref[...], (((1,), (1,)), ((), ())),
                            preferred_element_type=jnp.float32)
    g_ref[0] = h * dis[:, None]


def _conv1_kernel(a_ref, g_ref, gj_ref, dis_ref, b1_ref, w2_ref, out_ref):
    j = pl.program_id(1)
    i = pl.program_id(2)
    part = jax.lax.dot_general(a_ref[0], g_ref[0], (((0,), (0,)), ((), ())),
                               preferred_element_type=jnp.float32)

    @pl.when(i == 0)
    def _():
        out_ref[0] = part

    @pl.when(i > 0)
    def _():
        out_ref[0] += part

    @pl.when(i == NI - 1)
    def _():
        dis_j = dis_ref[0, 0, pl.ds(j * BJ, BJ)]
        h1c = jnp.maximum((out_ref[0] + gj_ref[0]) * dis_j[:, None]
                          + b1_ref[...], 0.0)
        h2 = jax.lax.dot_general(h1c, w2_ref[...], (((1,), (1,)), ((), ())),
                                 preferred_element_type=jnp.float32)
        out_ref[0] = h2 * dis_j[:, None]


def _conv2_kernel(a_ref, g_ref, gj_ref, dis_ref, b2_ref, wfc_ref, bfc_ref,
                  out_ref, acc_ref):
    j = pl.program_id(1)
    i = pl.program_id(2)
    part = jax.lax.dot_general(a_ref[0], g_ref[0], (((0,), (0,)), ((), ())),
                               preferred_element_type=jnp.float32)

    @pl.when(i == 0)
    def _():
        acc_ref[...] = part

    @pl.when(i > 0)
    def _():
        acc_ref[...] += part

    @pl.when(i == NI - 1)
    def _():
        dis_j = dis_ref[0, 0, pl.ds(j * BJ, BJ)]
        h2c = jnp.maximum((acc_ref[...] + gj_ref[0]) * dis_j[:, None]
                          + b2_ref[...], 0.0)
        of = jax.lax.dot_general(h2c, wfc_ref[...], (((1,), (1,)), ((), ())),
                                 preferred_element_type=jnp.float32)
        of = of + bfc_ref[...]
        out_ref[0] = jnp.where(dis_j[:, None] > 0, of, 0.0)


def _conv_call(a, g, dis, b, w_next, b_next, out_dim, second):
    a_spec = pl.BlockSpec((1, BI, BJ), lambda t, j, i: (t, i, j))
    g_spec = pl.BlockSpec((1, BI, HID), lambda t, j, i: (t, i, 0))
    gj_spec = pl.BlockSpec((1, BJ, HID), lambda t, j, i: (t, j, 0))
    dis_spec = pl.BlockSpec((1, 1, BN), lambda t, j, i: (t, 0, 0))
    b_spec = pl.BlockSpec((1, HID), lambda t, j, i: (0, 0))
    w_spec = pl.BlockSpec(w_next.shape, lambda t, j, i: (0, 0))
    out_spec = pl.BlockSpec((1, BJ, out_dim), lambda t, j, i: (t, j, 0))
    grid = (T, NJ, NI)
    params = pltpu.CompilerParams(
        dimension_semantics=("parallel", "parallel", "arbitrary"))
    if second:
        bfc_spec = pl.BlockSpec((1, out_dim), lambda t, j, i: (0, 0))
        return pl.pallas_call(
            _conv2_kernel, grid=grid,
            in_specs=[a_spec, g_spec, gj_spec, dis_spec, b_spec, w_spec,
                      bfc_spec],
            out_specs=out_spec,
            out_shape=jax.ShapeDtypeStruct((T, BN, out_dim), jnp.float32),
            scratch_shapes=[pltpu.VMEM((BJ, HID), jnp.float32)],
            compiler_params=params,
        )(a, g, g, dis, b, w_next, b_next)
    return pl.pallas_call(
        _conv1_kernel, grid=grid,
        in_specs=[a_spec, g_spec, gj_spec, dis_spec, b_spec, w_spec],
        out_specs=out_spec,
        out_shape=jax.ShapeDtypeStruct((T, BN, out_dim), jnp.float32),
        compiler_params=params,
    )(a, g, g, dis, b, w_next)


def kernel(big_batch_positions, big_batched_adjacency_pruned, ego_mask_batch,
           W1, b1, W2, b2, Wfc, bfc):
    x = big_batch_positions
    A = big_batched_adjacency_pruned
    m = jnp.transpose(ego_mask_batch, (1, 0, 2)).reshape(T, 1, BN)
    m = m.astype(jnp.float32)

    deg = pl.pallas_call(
        _deg_kernel, grid=(T, NI),
        in_specs=[pl.BlockSpec((1, 1, BN), lambda t, i: (t, 0, 0)),
                  pl.BlockSpec((1, BI, BN), lambda t, i: (t, i, 0))],
        out_specs=pl.BlockSpec((1, 1, BN), lambda t, i: (t, 0, 0)),
        out_shape=jax.ShapeDtypeStruct((T, 1, BN), jnp.float32),
        compiler_params=pltpu.CompilerParams(
            dimension_semantics=("parallel", "arbitrary")),
    )(m, A)

    dis, g1 = pl.pallas_call(
        _g1_kernel, grid=(T,),
        in_specs=[pl.BlockSpec((1, 1, BN), lambda t: (t, 0, 0)),
                  pl.BlockSpec((1, 1, BN), lambda t: (t, 0, 0)),
                  pl.BlockSpec((1, BN, IN_DIM), lambda t: (t, 0, 0)),
                  pl.BlockSpec((HID, IN_DIM), lambda t: (0, 0))],
        out_specs=[pl.BlockSpec((1, 1, BN), lambda t: (t, 0, 0)),
                   pl.BlockSpec((1, BN, HID), lambda t: (t, 0, 0))],
        out_shape=[jax.ShapeDtypeStruct((T, 1, BN), jnp.float32),
                   jax.ShapeDtypeStruct((T, BN, HID), jnp.float32)],
    )(deg, m, x, W1)

    b1r = b1.reshape(1, HID)
    b2r = b2.reshape(1, HID)
    bfcr = bfc.reshape(1, OUT)

    g2 = _conv_call(A, g1, dis, b1r, W2, None, HID, second=False)
    out = _conv_call(A, g2, dis, b2r, Wfc, bfcr, OUT, second=True)

    h_stack = out.reshape(T, B, N, OUT)
    return jnp.transpose(h_stack, (1, 2, 0, 3))


# speedup vs baseline: 2.4841x; 2.4841x over previous
import jax
import jax.numpy as jnp
from jax.experimental import pallas as pl
from jax.experimental.pallas import tpu as pltpu

T, B, N = 4, 8, 512
BN = B * N
IN_DIM, HID, OUT = 128, 128, 64

BI = 512
NI = BN // BI


def _deg_kernel(m_ref, a_ref, deg_ref, a8_ref):
    i = pl.program_id(1)
    a = a_ref[0]
    mi = m_ref[0, 0, pl.ds(i * BI, BI)]
    part = jnp.dot(mi[None, :], a, preferred_element_type=jnp.float32)
    a8_ref[0] = a.astype(jnp.int8)

    @pl.when(i == 0)
    def _():
        deg_ref[0] = part

    @pl.when(i > 0)
    def _():
        deg_ref[0] += part


def _g1_kernel(deg_ref, m_ref, x_ref, w1_ref, dis_ref, g_ref):
    m = m_ref[0, 0]
    deg = deg_ref[0, 0] * m + m
    dis = jnp.where(deg > 0, jax.lax.rsqrt(deg), 0.0)
    dis_ref[0, 0] = dis
    h = jax.lax.dot_general(x_ref[0], w1_ref[...], (((1,), (1,)), ((), ())),
                            preferred_element_type=jnp.float32)
    g_ref[0] = h * dis[:, None]


def _conv1_kernel(a8_ref, g_ref, dis_ref, b1_ref, w2_ref, out_ref, acc_ref):
    i = pl.program_id(1)
    a = a8_ref[0].astype(jnp.bfloat16)
    gb = g_ref[0, pl.ds(i * BI, BI), :].astype(jnp.bfloat16)
    part = jax.lax.dot_general(a, gb, (((0,), (0,)), ((), ())),
                               preferred_element_type=jnp.float32)

    @pl.when(i == 0)
    def _():
        acc_ref[...] = part

    @pl.when(i > 0)
    def _():
        acc_ref[...] += part

    @pl.when(i == NI - 1)
    def _():
        dis = dis_ref[0, 0]
        h1c = jnp.maximum((acc_ref[...] + g_ref[0]) * dis[:, None]
                          + b1_ref[...], 0.0)
        h2 = jax.lax.dot_general(h1c, w2_ref[...], (((1,), (1,)), ((), ())),
                                 preferred_element_type=jnp.float32)
        out_ref[0] = h2 * dis[:, None]


def _conv2_kernel(a8_ref, g_ref, dis_ref, b2_ref, wfc_ref, bfc_ref,
                  out_ref, acc_ref):
    i = pl.program_id(1)
    a = a8_ref[0].astype(jnp.bfloat16)
    gb = g_ref[0, pl.ds(i * BI, BI), :].astype(jnp.bfloat16)
    part = jax.lax.dot_general(a, gb, (((0,), (0,)), ((), ())),
                               preferred_element_type=jnp.float32)

    @pl.when(i == 0)
    def _():
        acc_ref[...] = part

    @pl.when(i > 0)
    def _():
        acc_ref[...] += part

    @pl.when(i == NI - 1)
    def _():
        dis = dis_ref[0, 0]
        h2c = jnp.maximum((acc_ref[...] + g_ref[0]) * dis[:, None]
                          + b2_ref[...], 0.0)
        of = jax.lax.dot_general(h2c, wfc_ref[...], (((1,), (1,)), ((), ())),
                                 preferred_element_type=jnp.float32)
        of = of + bfc_ref[...]
        out_ref[0] = jnp.where(dis[:, None] > 0, of, 0.0)


def _conv_call(kern, a8, g, dis, b, w_next, extra, out_dim):
    a_spec = pl.BlockSpec((1, BI, BN), lambda t, i: (t, i, 0))
    g_spec = pl.BlockSpec((1, BN, HID), lambda t, i: (t, 0, 0))
    dis_spec = pl.BlockSpec((1, 1, BN), lambda t, i: (t, 0, 0))
    b_spec = pl.BlockSpec((1, HID), lambda t, i: (0, 0))
    w_spec = pl.BlockSpec(w_next.shape, lambda t, i: (0, 0))
    out_spec = pl.BlockSpec((1, BN, out_dim), lambda t, i: (t, 0, 0))
    in_specs = [a_spec, g_spec, dis_spec, b_spec, w_spec]
    args = [a8, g, dis, b, w_next]
    if extra is not None:
        in_specs.append(pl.BlockSpec((1, out_dim), lambda t, i: (0, 0)))
        args.append(extra)
    return pl.pallas_call(
        kern, grid=(T, NI),
        in_specs=in_specs,
        out_specs=out_spec,
        out_shape=jax.ShapeDtypeStruct((T, BN, out_dim), jnp.float32),
        scratch_shapes=[pltpu.VMEM((BN, HID), jnp.float32)],
        compiler_params=pltpu.CompilerParams(
            dimension_semantics=("parallel", "arbitrary")),
    )(*args)


def kernel(big_batch_positions, big_batched_adjacency_pruned, ego_mask_batch,
           W1, b1, W2, b2, Wfc, bfc):
    x = big_batch_positions
    A = big_batched_adjacency_pruned
    m = jnp.transpose(ego_mask_batch, (1, 0, 2)).reshape(T, 1, BN)
    m = m.astype(jnp.float32)

    deg, A8 = pl.pallas_call(
        _deg_kernel, grid=(T, NI),
        in_specs=[pl.BlockSpec((1, 1, BN), lambda t, i: (t, 0, 0)),
                  pl.BlockSpec((1, BI, BN), lambda t, i: (t, i, 0))],
        out_specs=[pl.BlockSpec((1, 1, BN), lambda t, i: (t, 0, 0)),
                   pl.BlockSpec((1, BI, BN), lambda t, i: (t, i, 0))],
        out_shape=[jax.ShapeDtypeStruct((T, 1, BN), jnp.float32),
                   jax.ShapeDtypeStruct((T, BN, BN), jnp.int8)],
        compiler_params=pltpu.CompilerParams(
            dimension_semantics=("parallel", "arbitrary")),
    )(m, A)

    dis, g1 = pl.pallas_call(
        _g1_kernel, grid=(T,),
        in_specs=[pl.BlockSpec((1, 1, BN), lambda t: (t, 0, 0)),
                  pl.BlockSpec((1, 1, BN), lambda t: (t, 0, 0)),
                  pl.BlockSpec((1, BN, IN_DIM), lambda t: (t, 0, 0)),
                  pl.BlockSpec((HID, IN_DIM), lambda t: (0, 0))],
        out_specs=[pl.BlockSpec((1, 1, BN), lambda t: (t, 0, 0)),
                   pl.BlockSpec((1, BN, HID), lambda t: (t, 0, 0))],
        out_shape=[jax.ShapeDtypeStruct((T, 1, BN), jnp.float32),
                   jax.ShapeDtypeStruct((T, BN, HID), jnp.float32)],
    )(deg, m, x, W1)

    b1r = b1.reshape(1, HID)
    b2r = b2.reshape(1, HID)
    bfcr = bfc.reshape(1, OUT)

    g2 = _conv_call(_conv1_kernel, A8, g1, dis, b1r, W2, None, HID)
    out = _conv_call(_conv2_kernel, A8, g2, dis, b2r, Wfc, bfcr, OUT)

    h_stack = out.reshape(T, B, N, OUT)
    return jnp.transpose(h_stack, (1, 2, 0, 3))


# speedup vs baseline: 2.8041x; 1.1288x over previous
import jax
import jax.numpy as jnp
from jax.experimental import pallas as pl
from jax.experimental.pallas import tpu as pltpu

T, B, N = 4, 8, 512
BN = B * N
IN_DIM, HID, OUT = 128, 128, 64

BI = 512
NI = BN // BI


def _fused_kernel(m_ref, a_ref, x_ref, w1_ref, b1_ref, w2_ref, b2_ref,
                  wfc_ref, bfc_ref, out_ref, a8v, acc, g, deg, dis):
    j = pl.program_id(1)
    phase = j // NI
    i = j % NI

    @pl.when(phase == 0)
    def _():
        a = a_ref[0]
        a8v[pl.ds(i * BI, BI), :] = a.astype(jnp.bfloat16)
        mi = m_ref[0, 0, pl.ds(i * BI, BI)]
        part = jnp.dot(mi[None, :], a, preferred_element_type=jnp.float32)

        @pl.when(i == 0)
        def _():
            deg[...] = part

        @pl.when(i > 0)
        def _():
            deg[...] += part

    @pl.when(phase == 1)
    def _():
        @pl.when(i == 0)
        def _():
            m = m_ref[0, 0]
            d = deg[0] * m + m
            dis[...] = jnp.where(d > 0, jax.lax.rsqrt(d), 0.0)[None]
            h = jax.lax.dot_general(x_ref[0], w1_ref[...],
                                    (((1,), (1,)), ((), ())),
                                    preferred_element_type=jnp.float32)
            g[...] = h * dis[0][:, None]

        ab = a8v[pl.ds(i * BI, BI), :]
        gb = g[pl.ds(i * BI, BI), :].astype(jnp.bfloat16)
        part = jax.lax.dot_general(ab, gb, (((0,), (0,)), ((), ())),
                                   preferred_element_type=jnp.float32)

        @pl.when(i == 0)
        def _():
            acc[...] = part

        @pl.when(i > 0)
        def _():
            acc[...] += part

        @pl.when(i == NI - 1)
        def _():
            d = dis[0]
            h1c = jnp.maximum((acc[...] + g[...]) * d[:, None]
                              + b1_ref[...], 0.0)
            h2 = jax.lax.dot_general(h1c, w2_ref[...], (((1,), (1,)), ((), ())),
                                     preferred_element_type=jnp.float32)
            g[...] = h2 * d[:, None]

    @pl.when(phase == 2)
    def _():
        ab = a8v[pl.ds(i * BI, BI), :]
        gb = g[pl.ds(i * BI, BI), :].astype(jnp.bfloat16)
        part = jax.lax.dot_general(ab, gb, (((0,), (0,)), ((), ())),
                                   preferred_element_type=jnp.float32)

        @pl.when(i == 0)
        def _():
            acc[...] = part

        @pl.when(i > 0)
        def _():
            acc[...] += part

        @pl.when(i == NI - 1)
        def _():
            d = dis[0]
            h2c = jnp.maximum((acc[...] + g[...]) * d[:, None]
                              + b2_ref[...], 0.0)
            of = jax.lax.dot_general(h2c, wfc_ref[...], (((1,), (1,)), ((), ())),
                                     preferred_element_type=jnp.float32)
            of = of + bfc_ref[...]
            out_ref[0] = jnp.where(d[:, None] > 0, of, 0.0)


def kernel(big_batch_positions, big_batched_adjacency_pruned, ego_mask_batch,
           W1, b1, W2, b2, Wfc, bfc):
    x = big_batch_positions
    A = big_batched_adjacency_pruned
    m = jnp.transpose(ego_mask_batch, (1, 0, 2)).reshape(T, 1, BN)
    m = m.astype(jnp.float32)
    b1r = b1.reshape(1, HID)
    b2r = b2.reshape(1, HID)
    bfcr = bfc.reshape(1, OUT)

    out = pl.pallas_call(
        _fused_kernel, grid=(T, 3 * NI),
        in_specs=[
            pl.BlockSpec((1, 1, BN), lambda t, j: (t, 0, 0)),
            pl.BlockSpec((1, BI, BN),
                         lambda t, j: (t, jnp.minimum(j, NI - 1), 0)),
            pl.BlockSpec((1, BN, IN_DIM), lambda t, j: (t, 0, 0)),
            pl.BlockSpec((HID, IN_DIM), lambda t, j: (0, 0)),
            pl.BlockSpec((1, HID), lambda t, j: (0, 0)),
            pl.BlockSpec((HID, HID), lambda t, j: (0, 0)),
            pl.BlockSpec((1, HID), lambda t, j: (0, 0)),
            pl.BlockSpec((OUT, HID), lambda t, j: (0, 0)),
            pl.BlockSpec((1, OUT), lambda t, j: (0, 0)),
        ],
        out_specs=pl.BlockSpec((1, BN, OUT), lambda t, j: (t, 0, 0)),
        out_shape=jax.ShapeDtypeStruct((T, BN, OUT), jnp.float32),
        scratch_shapes=[
            pltpu.VMEM((BN, BN), jnp.bfloat16),
            pltpu.VMEM((BN, HID), jnp.float32),
            pltpu.VMEM((BN, HID), jnp.float32),
            pltpu.VMEM((1, BN), jnp.float32),
            pltpu.VMEM((1, BN), jnp.float32),
        ],
        compiler_params=pltpu.CompilerParams(
            dimension_semantics=("arbitrary", "arbitrary"),
            vmem_limit_bytes=100 * 1024 * 1024),
    )(m, A, x, W1, b1r, W2, b2r, Wfc, bfcr)

    h_stack = out.reshape(T, B, N, OUT)
    return jnp.transpose(h_stack, (1, 2, 0, 3))
